# Initial kernel scaffold; baseline (speedup 1.0000x reference)
import jax
import jax.numpy as jnp
from jax.experimental import pallas as pl


def kernel(pred, gt):
    raise NotImplementedError("write your pallas kernel here")


# speedup vs baseline: 1.0684x; 1.0684x over previous
import jax
import jax.numpy as jnp
from jax.experimental import pallas as pl
from jax.experimental.pallas import tpu as pltpu

_TN = 512


def _chamfer_tc_kernel(pred_ref, gtt_ref, out_ref, d2_acc, s1_acc, s2_acc):
    b = pl.program_id(0)
    t = pl.program_id(1)
    nb = pl.num_programs(0)
    nt = pl.num_programs(1)

    p = pred_ref[0]
    g = gtt_ref[0]

    a2 = jnp.sum(p * p, axis=1, keepdims=True)
    b2 = jnp.sum(g * g, axis=0, keepdims=True)
    ab = jax.lax.dot_general(
        p.astype(jnp.bfloat16),
        g.astype(jnp.bfloat16),
        (((1,), (0,)), ((), ())),
        preferred_element_type=jnp.float32,
    )
    d = jnp.maximum(a2 + b2 - 2.0 * ab, 0.0)

    @pl.when((b == 0) & (t == 0))
    def _init_sums():
        s1_acc[...] = jnp.zeros_like(s1_acc)
        s2_acc[...] = jnp.zeros_like(s2_acc)

    @pl.when(t == 0)
    def _init_colmin():
        d2_acc[...] = jnp.full_like(d2_acc, jnp.inf)

    row_min = jnp.min(d, axis=1, keepdims=True)
    s1_acc[...] += jnp.sum(row_min, keepdims=True)

    d2_acc[...] = jnp.minimum(d2_acc[...], jnp.min(d, axis=0, keepdims=True))

    @pl.when(t == nt - 1)
    def _fold_batch():
        s2_acc[...] += jnp.sum(d2_acc[...], keepdims=True)

    @pl.when((b == nb - 1) & (t == nt - 1))
    def _emit():
        n_total = nb * nt * _TN
        m_total = d2_acc.shape[1] * nb
        out_ref[...] = (s1_acc[...] / n_total) + (s2_acc[...] / m_total)


def kernel(pred, gt):
    B, N, _ = pred.shape
    M = gt.shape[1]
    gtt = jnp.swapaxes(gt, 1, 2)
    nt = N // _TN

    out = pl.pallas_call(
        _chamfer_tc_kernel,
        grid=(B, nt),
        in_specs=[
            pl.BlockSpec((1, _TN, 3), lambda b, t: (b, t, 0)),
            pl.BlockSpec((1, 3, M), lambda b, t: (b, 0, 0)),
        ],
        out_specs=pl.BlockSpec((1, 1), lambda b, t: (0, 0)),
        out_shape=jax.ShapeDtypeStruct((1, 1), jnp.float32),
        scratch_shapes=[
            pltpu.VMEM((1, M), jnp.float32),
            pltpu.VMEM((1, 1), jnp.float32),
            pltpu.VMEM((1, 1), jnp.float32),
        ],
    )(pred, gtt)
    return out[0, 0]


# speedup vs baseline: 1.0793x; 1.0102x over previous
import jax
import jax.numpy as jnp
from jax.experimental import pallas as pl
from jax.experimental.pallas import tpu as pltpu

_TN = 512


def _chamfer_tc_kernel(pred_ref, gtt_ref, out_ref, d2_acc, s1_acc, s2_acc):
    b = pl.program_id(0)
    t = pl.program_id(1)
    nb = pl.num_programs(0)
    nt = pl.num_programs(1)

    p = pred_ref[0]
    g = gtt_ref[0]

    a2 = jnp.sum(p * p, axis=1, keepdims=True)
    b2 = jnp.sum(g * g, axis=0, keepdims=True)
    ab = jax.lax.dot_general(
        p.astype(jnp.bfloat16),
        g.astype(jnp.bfloat16),
        (((1,), (0,)), ((), ())),
        preferred_element_type=jnp.float32,
    )
    d = (ab * -2.0 + b2) + a2

    @pl.when((b == 0) & (t == 0))
    def _init_sums():
        s1_acc[...] = jnp.zeros_like(s1_acc)
        s2_acc[...] = jnp.zeros_like(s2_acc)

    @pl.when(t == 0)
    def _init_colmin():
        d2_acc[...] = jnp.full_like(d2_acc, jnp.inf)

    row_min = jnp.maximum(jnp.min(d, axis=1, keepdims=True), 0.0)
    s1_acc[...] += jnp.sum(row_min, keepdims=True)

    d2_acc[...] = jnp.minimum(d2_acc[...], jnp.min(d, axis=0, keepdims=True))

    @pl.when(t == nt - 1)
    def _fold_batch():
        s2_acc[...] += jnp.sum(jnp.maximum(d2_acc[...], 0.0), keepdims=True)

    @pl.when((b == nb - 1) & (t == nt - 1))
    def _emit():
        n_total = nb * nt * _TN
        m_total = d2_acc.shape[1] * nb
        out_ref[...] = (s1_acc[...] / n_total) + (s2_acc[...] / m_total)


def kernel(pred, gt):
    B, N, _ = pred.shape
    M = gt.shape[1]
    gtt = jnp.swapaxes(gt, 1, 2)
    nt = N // _TN

    out = pl.pallas_call(
        _chamfer_tc_kernel,
        grid=(B, nt),
        in_specs=[
            pl.BlockSpec((1, _TN, 3), lambda b, t: (b, t, 0)),
            pl.BlockSpec((1, 3, M), lambda b, t: (b, 0, 0)),
        ],
        out_specs=pl.BlockSpec((1, 1), lambda b, t: (0, 0)),
        out_shape=jax.ShapeDtypeStruct((1, 1), jnp.float32),
        scratch_shapes=[
            pltpu.VMEM((1, M), jnp.float32),
            pltpu.VMEM((1, 1), jnp.float32),
            pltpu.VMEM((1, 1), jnp.float32),
        ],
    )(pred, gtt)
    return out[0, 0]


# speedup vs baseline: 1.2907x; 1.1958x over previous
import jax
import jax.numpy as jnp
from jax.experimental import pallas as pl
from jax.experimental.pallas import tpu as pltpu

_TN = 512
_NC = 8


def _chamfer_tc_kernel(pred_ref, gs_ref, out_ref, d2_acc, s1_acc, s2_acc):
    b = pl.program_id(0)
    t = pl.program_id(1)
    nb = pl.num_programs(0)
    nt = pl.num_programs(1)

    p = pred_ref[0]
    gs = gs_ref[0]

    a2 = jnp.sum(p * p, axis=1, keepdims=True)
    b2 = 0.25 * jnp.sum(gs * gs, axis=0, keepdims=True)
    pb = p.astype(jnp.bfloat16)
    gsb = gs.astype(jnp.bfloat16)

    @pl.when((b == 0) & (t == 0))
    def _init_sums():
        s1_acc[...] = jnp.zeros_like(s1_acc)
        s2_acc[...] = jnp.zeros_like(s2_acc)

    @pl.when(t == 0)
    def _init_colmin():
        d2_acc[...] = jnp.full_like(d2_acc, jnp.inf)

    M = gs.shape[1]
    cb = M // _NC
    row_min = None
    col_mins = []
    for c in range(_NC):
        lo, hi = c * cb, (c + 1) * cb
        ab2 = jax.lax.dot_general(
            pb, gsb[:, lo:hi],
            (((1,), (0,)), ((), ())),
            preferred_element_type=jnp.float32,
        )
        rm = jnp.min(ab2 + b2[:, lo:hi], axis=1, keepdims=True)
        row_min = rm if row_min is None else jnp.minimum(row_min, rm)
        col_mins.append(jnp.min(ab2 + a2, axis=0, keepdims=True))

    dist1 = jnp.maximum(row_min + a2, 0.0)
    s1_acc[...] += jnp.sum(dist1, keepdims=True)

    col_min = jnp.concatenate(col_mins, axis=1)
    d2_acc[...] = jnp.minimum(d2_acc[...], col_min)

    @pl.when(t == nt - 1)
    def _fold_batch():
        dist2 = jnp.maximum(d2_acc[...] + b2, 0.0)
        s2_acc[...] += jnp.sum(dist2, keepdims=True)

    @pl.when((b == nb - 1) & (t == nt - 1))
    def _emit():
        n_total = nb * nt * _TN
        m_total = d2_acc.shape[1] * nb
        out_ref[...] = (s1_acc[...] / n_total) + (s2_acc[...] / m_total)


def kernel(pred, gt):
    B, N, _ = pred.shape
    M = gt.shape[1]
    gs = -2.0 * jnp.swapaxes(gt, 1, 2)
    nt = N // _TN

    out = pl.pallas_call(
        _chamfer_tc_kernel,
        grid=(B, nt),
        in_specs=[
            pl.BlockSpec((1, _TN, 3), lambda b, t: (b, t, 0)),
            pl.BlockSpec((1, 3, M), lambda b, t: (b, 0, 0)),
        ],
        out_specs=pl.BlockSpec((1, 1), lambda b, t: (0, 0)),
        out_shape=jax.ShapeDtypeStruct((1, 1), jnp.float32),
        scratch_shapes=[
            pltpu.VMEM((1, M), jnp.float32),
            pltpu.VMEM((1, 1), jnp.float32),
            pltpu.VMEM((1, 1), jnp.float32),
        ],
    )(pred, gs)
    return out[0, 0]


# speedup vs baseline: 1.6725x; 1.2958x over previous
import jax
import jax.numpy as jnp
from jax.experimental import pallas as pl
from jax.experimental.pallas import tpu as pltpu

_NC = 8


def _chamfer_tc_kernel(pred_ref, gs_ref, out_ref, s1_acc, s2_acc):
    b = pl.program_id(0)
    nb = pl.num_programs(0)

    p = pred_ref[0]
    gs = gs_ref[0]
    N = p.shape[0]
    M = gs.shape[1]

    one_p = jnp.ones((N, 1), jnp.bfloat16)
    one_g = jnp.ones((1, M), jnp.bfloat16)

    a2 = jnp.sum(p * p, axis=1, keepdims=True)
    a2_hi = a2.astype(jnp.bfloat16)
    a2_lo = (a2 - a2_hi.astype(jnp.float32)).astype(jnp.bfloat16)
    b2 = 0.25 * jnp.sum(gs * gs, axis=0, keepdims=True)
    b2_hi = b2.astype(jnp.bfloat16)
    b2_lo = (b2 - b2_hi.astype(jnp.float32)).astype(jnp.bfloat16)

    p_aug = jnp.concatenate(
        [p.astype(jnp.bfloat16), a2_hi, a2_lo, one_p, one_p], axis=1)
    g_aug = jnp.concatenate(
        [gs.astype(jnp.bfloat16), one_g, one_g, b2_hi, b2_lo], axis=0)

    @pl.when(b == 0)
    def _init_sums():
        s1_acc[...] = jnp.zeros_like(s1_acc)
        s2_acc[...] = jnp.zeros_like(s2_acc)

    cb = M // _NC
    rm_acc = None
    col_mins = []
    for c in range(_NC):
        lo, hi = c * cb, (c + 1) * cb
        e = jax.lax.dot_general(
            p_aug, g_aug[:, lo:hi],
            (((1,), (0,)), ((), ())),
            preferred_element_type=jnp.float32,
        )
        for k in range(cb // 128):
            ek = e[:, k * 128:(k + 1) * 128]
            rm_acc = ek if rm_acc is None else jnp.minimum(rm_acc, ek)
        col_mins.append(jnp.min(e, axis=0, keepdims=True))

    row_min = jnp.min(rm_acc, axis=1, keepdims=True)
    s1_acc[...] += jnp.sum(jnp.maximum(row_min, 0.0), keepdims=True)

    col_min = jnp.concatenate(col_mins, axis=1)
    s2_acc[...] += jnp.sum(jnp.maximum(col_min, 0.0), keepdims=True)

    @pl.when(b == nb - 1)
    def _emit():
        out_ref[...] = (s1_acc[...] / (nb * N)) + (s2_acc[...] / (nb * M))


def kernel(pred, gt):
    B, N, _ = pred.shape
    M = gt.shape[1]
    gs = -2.0 * jnp.swapaxes(gt, 1, 2)

    out = pl.pallas_call(
        _chamfer_tc_kernel,
        grid=(B,),
        in_specs=[
            pl.BlockSpec((1, N, 3), lambda b: (b, 0, 0)),
            pl.BlockSpec((1, 3, M), lambda b: (b, 0, 0)),
        ],
        out_specs=pl.BlockSpec((1, 1), lambda b: (0, 0)),
        out_shape=jax.ShapeDtypeStruct((1, 1), jnp.float32),
        scratch_shapes=[
            pltpu.VMEM((1, 1), jnp.float32),
            pltpu.VMEM((1, 1), jnp.float32),
        ],
    )(pred, gs)
    return out[0, 0]
